# baseline (speedup 1.0000x reference)
import functools

import jax
import jax.numpy as jnp
from jax import lax
from jax.experimental import pallas as pl
from jax.experimental.pallas import tpu as pltpu
from jax.experimental.pallas import tpu_sc as plsc

N_NODES = 5000
DIM = 128
E = 160000
NP = 5120
ROWS_PER_TILE = NP // 16
NBUF = 5
N_TILES = 32
TE = 5120
EPAD = N_TILES * TE
CH = 64
NCH = TE // CH

_HI = jax.lax.Precision.HIGHEST


def _dot(a, b):
    return jnp.dot(a, b, precision=_HI, preferred_element_type=jnp.float32)


def _dense0_body(xu_ref, xi_ref, wuu_ref, wui_ref, wiu_ref, a_ref,
                 huu_ref, hui_ref, hiu_ref, ee_ref):
    xu = xu_ref[...]
    xi = xi_ref[...]
    huu = _dot(xu, wuu_ref[...])
    hui = _dot(xu, wui_ref[...])
    hiu = _dot(xi, wiu_ref[...])
    hdui = _dot(xi, wui_ref[...])
    hdiu = _dot(xu, wiu_ref[...])
    huu_ref[...] = huu
    hui_ref[...] = hui
    hiu_ref[...] = hiu
    a = a_ref[...]
    mv = lambda h, v: jnp.sum(h * v[None, :], axis=1)
    ee = jnp.stack([
        mv(huu, a[0]), mv(huu, a[1]),
        mv(hui, a[2]), mv(hdui, a[3]),
        mv(hiu, a[4]), mv(hdiu, a[5]),
        jnp.zeros((N_NODES,), jnp.float32),
        jnp.zeros((N_NODES,), jnp.float32),
    ])
    ee_ref[...] = jnp.concatenate(
        [ee, jnp.zeros((8, NP - N_NODES), jnp.float32)], axis=1)


def _agg(n_pair, s0, s1):
    num = n_pair[0, :N_NODES, :] + n_pair[1, :N_NODES, :]
    den = s0[:N_NODES] + s1[:N_NODES] + 1e-9
    return num / den[:, None]


def _combine_dense_body(nuu_ref, suu0_ref, suu1_ref, nui_ref, sui0_ref, sui1_ref,
                        niu_ref, siu0_ref, siu1_ref,
                        wuu_ref, wui_ref, wiu_ref, a_ref,
                        huu_ref, hui_ref, hiu_ref, ee_ref):
    mu = 0.5 * (_agg(nuu_ref[...], suu0_ref[...], suu1_ref[...])
                + _agg(niu_ref[...], siu0_ref[...], siu1_ref[...]))
    mi = _agg(nui_ref[...], sui0_ref[...], sui1_ref[...])
    xu = jnp.where(mu > 0, mu, jnp.exp(jnp.minimum(mu, 0.0)) - 1.0)
    xi = jnp.where(mi > 0, mi, jnp.exp(jnp.minimum(mi, 0.0)) - 1.0)
    _dense0_body(_Val(xu), _Val(xi), wuu_ref, wui_ref, wiu_ref, a_ref,
                 huu_ref, hui_ref, hiu_ref, ee_ref)


class _Val:
    def __init__(self, v):
        self._v = v

    def __getitem__(self, idx):
        return self._v


def _final_body(nuu_ref, suu0_ref, suu1_ref, nui_ref, sui0_ref, sui1_ref,
                niu_ref, siu0_ref, siu1_ref,
                wc_ref, bc_ref, out_ref):
    mu = 0.5 * (_agg(nuu_ref[...], suu0_ref[...], suu1_ref[...])
                + _agg(niu_ref[...], siu0_ref[...], siu1_ref[...]))
    mi = _agg(nui_ref[...], sui0_ref[...], sui1_ref[...])
    hg = jnp.mean(mu, axis=0) + jnp.mean(mi, axis=0)
    val = jnp.sum(hg * wc_ref[...][:, 0]) + bc_ref[...][0]
    out_ref[...] = jax.nn.sigmoid(val).reshape(1)


_dense0 = pl.pallas_call(
    _dense0_body,
    out_shape=(
        jax.ShapeDtypeStruct((N_NODES, DIM), jnp.float32),
        jax.ShapeDtypeStruct((N_NODES, DIM), jnp.float32),
        jax.ShapeDtypeStruct((N_NODES, DIM), jnp.float32),
        jax.ShapeDtypeStruct((8, NP), jnp.float32),
    ),
)

_combine_dense = pl.pallas_call(
    _combine_dense_body,
    out_shape=(
        jax.ShapeDtypeStruct((N_NODES, DIM), jnp.float32),
        jax.ShapeDtypeStruct((N_NODES, DIM), jnp.float32),
        jax.ShapeDtypeStruct((N_NODES, DIM), jnp.float32),
        jax.ShapeDtypeStruct((8, NP), jnp.float32),
    ),
)

_final = pl.pallas_call(
    _final_body,
    out_shape=jax.ShapeDtypeStruct((1,), jnp.float32),
)


_sc_mesh = plsc.VectorSubcoreMesh(core_axis_name="c", subcore_axis_name="s")

import dataclasses as _dataclasses

_sc_params = pltpu.CompilerParams()
if "needs_layout_passes" in pltpu.CompilerParams.__dataclass_fields__:
    _sc_params = _dataclasses.replace(_sc_params, needs_layout_passes=False)


@functools.partial(
    pl.kernel,
    out_type=(
        jax.ShapeDtypeStruct((2, NP, DIM), jnp.float32),
        jax.ShapeDtypeStruct((NP,), jnp.float32),
        jax.ShapeDtypeStruct((NP,), jnp.float32),
    ),
    mesh=_sc_mesh,
    compiler_params=_sc_params,
    scratch_types=[
        pltpu.VMEM((NCH, CH), jnp.int32),
        pltpu.VMEM((NCH, CH), jnp.int32),
        pltpu.VMEM((NP,), jnp.float32),
        pltpu.VMEM((NP,), jnp.float32),
        pltpu.VMEM((TE,), jnp.float32),
        pltpu.VMEM((NBUF, CH, DIM), jnp.float32),
        pltpu.VMEM_SHARED((NP, DIM), jnp.float32),
        pltpu.VMEM_SHARED((NP,), jnp.float32),
    ] + [pltpu.SemaphoreType.DMA] * (2 * NBUF),
)
def _rel_edges(h_hbm, es_hbm, ed_hbm, src_hbm, dst_hbm,
               n_out, s0_out, s1_out,
               src_v, dst_v, es_v, ed_v, ex_v, bufs, n_acc, s_acc, *sems):
    gsems = sems[:NBUF]
    ssems = sems[NBUF:]
    cid = lax.axis_index("c")
    sid = lax.axis_index("s")
    wid = sid * 2 + cid

    pltpu.sync_copy(src_hbm.at[wid], src_v)
    pltpu.sync_copy(dst_hbm.at[wid], dst_v)
    pltpu.sync_copy(es_hbm, es_v)
    pltpu.sync_copy(ed_hbm, ed_v)

    zeros16 = jnp.zeros((16,), jnp.float32)

    @pl.loop(0, CH)
    def _(r):
        for k in range(DIM // 16):
            bufs[0, r, pl.ds(k * 16, 16)] = zeros16

    @pl.loop(0, TE, step=16)
    def _(i):
        ex_v[pl.ds(i, 16)] = zeros16

    base = sid * ROWS_PER_TILE
    buf0 = bufs.at[0]
    for j in range(ROWS_PER_TILE // CH):
        pltpu.sync_copy(buf0, n_acc.at[pl.ds(base + j * CH, CH)])

    @pl.when(sid == 0)
    def _():
        pltpu.sync_copy(ex_v.at[pl.ds(0, NP)], s_acc)

    for b in range(NBUF):
        pltpu.async_copy(h_hbm.at[src_v.at[b]], bufs.at[b], gsems[b])

    @pl.loop(0, NCH)
    def _(c):
        @pl.loop(0, CH, step=16)
        def _(j):
            s16 = src_v[c, pl.ds(j, 16)]
            d16 = dst_v[c, pl.ds(j, 16)]
            logit = plsc.load_gather(es_v, [s16]) + plsc.load_gather(ed_v, [d16])
            e = jnp.maximum(logit, 0.2 * logit)
            ex_v[pl.ds(c * CH + j, 16)] = jnp.exp(e)

    plsc.subcore_barrier()

    def _wait_gather(b):
        pltpu.make_async_copy(h_hbm.at[src_v.at[0]], bufs.at[b], gsems[b]).wait()

    def _wait_scatter(b):
        pltpu.make_async_copy(bufs.at[b], n_acc.at[dst_v.at[0]], ssems[b]).wait()
        pltpu.make_async_copy(ex_v.at[pl.ds(0, CH)], s_acc.at[dst_v.at[0]],
                              ssems[b]).wait()

    @pl.loop(0, NCH, step=NBUF)
    def _(c0):
        for b in range(NBUF):
            cc = c0 + b
            _wait_gather(b)

            @pl.loop(0, CH, step=4)
            def _(r0):
                for u in range(4):
                    r = r0 + u
                    w = plsc.load_gather(
                        ex_v, [jnp.full((16,), cc * CH + r, jnp.int32)])
                    for k in range(DIM // 16):
                        bufs[b, r, pl.ds(k * 16, 16)] = (
                            bufs[b, r, pl.ds(k * 16, 16)] * w)

            pltpu.async_copy(bufs.at[b], n_acc.at[dst_v.at[cc]], ssems[b], add=True)
            pltpu.async_copy(ex_v.at[pl.ds(cc * CH, CH)], s_acc.at[dst_v.at[cc]],
                             ssems[b], add=True)

            @pl.when(cc + NBUF < NCH)
            def _():
                _wait_scatter(b)
                pltpu.async_copy(h_hbm.at[src_v.at[cc + NBUF]], bufs.at[b],
                                 gsems[b])

    for b in range(NBUF):
        _wait_scatter(b)

    plsc.subcore_barrier()

    pltpu.sync_copy(n_acc.at[pl.ds(base, ROWS_PER_TILE)],
                    n_out.at[cid, pl.ds(base, ROWS_PER_TILE)])

    @pl.when((sid == 0) & (cid == 0))
    def _():
        pltpu.sync_copy(s_acc, s0_out)

    @pl.when((sid == 0) & (cid == 1))
    def _():
        pltpu.sync_copy(s_acc, s1_out)


def _prep_edges(ei):
    pad = EPAD - E
    src = jnp.concatenate(
        [ei[0], (jnp.arange(pad, dtype=jnp.int32) % N_NODES)])
    dst = jnp.concatenate(
        [ei[1], N_NODES + (jnp.arange(pad, dtype=jnp.int32) % 8)])
    return src.reshape(N_TILES, NCH, CH), dst.reshape(N_TILES, NCH, CH)


def kernel(x_user, x_item, edge_uu, edge_ui, edge_iu,
           W_0_uu, as_0_uu, ad_0_uu, W_0_ui, as_0_ui, ad_0_ui,
           W_0_iu, as_0_iu, ad_0_iu, W_1_uu, as_1_uu, ad_1_uu,
           W_1_ui, as_1_ui, ad_1_ui, W_1_iu, as_1_iu, ad_1_iu,
           Wc, bc):
    suu, duu = _prep_edges(edge_uu)
    sui, dui = _prep_edges(edge_ui)
    siu, diu = _prep_edges(edge_iu)

    a0 = jnp.stack([as_0_uu, ad_0_uu, as_0_ui, ad_0_ui, as_0_iu, ad_0_iu])
    a1 = jnp.stack([as_1_uu, ad_1_uu, as_1_ui, ad_1_ui, as_1_iu, ad_1_iu])

    huu, hui, hiu, ee = _dense0(x_user, x_item, W_0_uu, W_0_ui, W_0_iu, a0)
    nuu, suu0, suu1 = _rel_edges(huu, ee[0], ee[1], suu, duu)
    nui, sui0, sui1 = _rel_edges(hui, ee[2], ee[3], sui, dui)
    niu, siu0, siu1 = _rel_edges(hiu, ee[4], ee[5], siu, diu)

    huu, hui, hiu, ee = _combine_dense(
        nuu, suu0, suu1, nui, sui0, sui1, niu, siu0, siu1,
        W_1_uu, W_1_ui, W_1_iu, a1)
    nuu, suu0, suu1 = _rel_edges(huu, ee[0], ee[1], suu, duu)
    nui, sui0, sui1 = _rel_edges(hui, ee[2], ee[3], sui, dui)
    niu, siu0, siu1 = _rel_edges(hiu, ee[4], ee[5], siu, diu)

    return _final(nuu, suu0, suu1, nui, sui0, sui1, niu, siu0, siu1, Wc, bc)


# speedup vs baseline: 1.0376x; 1.0376x over previous
import functools

import jax
import jax.numpy as jnp
from jax import lax
from jax.experimental import pallas as pl
from jax.experimental.pallas import tpu as pltpu
from jax.experimental.pallas import tpu_sc as plsc

N_NODES = 5000
DIM = 128
E = 160000
NP = 5120
NP2 = 2 * NP
ROWS2_PER_TILE = NP2 // 16
NBUF = 2
N_TILES = 32
TE = 5120
EPAD = N_TILES * TE
CH = 64
NCH = TE // CH

_HI = jax.lax.Precision.HIGHEST


def _dot(a, b):
    return jnp.dot(a, b, precision=_HI, preferred_element_type=jnp.float32)


def _make_vtab(h, es):
    pad = jnp.zeros((NP - N_NODES, DIM), jnp.float32)
    return jnp.concatenate([
        jnp.exp(es)[:, None] * h, pad,
        jnp.exp(0.2 * es)[:, None] * h, pad,
    ], axis=0)


def _dense0_body(xu_ref, xi_ref, wuu_ref, wui_ref, wiu_ref, a_ref,
                 vuu_ref, vui_ref, viu_ref, ee_ref):
    xu = xu_ref[...]
    xi = xi_ref[...]
    huu = _dot(xu, wuu_ref[...])
    hui = _dot(xu, wui_ref[...])
    hiu = _dot(xi, wiu_ref[...])
    hdui = _dot(xi, wui_ref[...])
    hdiu = _dot(xu, wiu_ref[...])
    a = a_ref[...]
    mv = lambda h, v: jnp.sum(h * v[None, :], axis=1)
    es_uu = mv(huu, a[0])
    es_ui = mv(hui, a[2])
    es_iu = mv(hiu, a[4])
    vuu_ref[...] = _make_vtab(huu, es_uu)
    vui_ref[...] = _make_vtab(hui, es_ui)
    viu_ref[...] = _make_vtab(hiu, es_iu)
    ee = jnp.stack([
        es_uu, mv(huu, a[1]),
        es_ui, mv(hdui, a[3]),
        es_iu, mv(hdiu, a[5]),
        jnp.zeros((N_NODES,), jnp.float32),
        jnp.zeros((N_NODES,), jnp.float32),
    ])
    ee_ref[...] = jnp.concatenate(
        [ee, jnp.zeros((8, NP - N_NODES), jnp.float32)], axis=1)


def _agg(n_pair, s0, s1, ed):
    bp = jnp.exp(ed[:N_NODES])
    bn = jnp.exp(0.2 * ed[:N_NODES])
    nsum_p = n_pair[0, :N_NODES, :] + n_pair[1, :N_NODES, :]
    nsum_n = n_pair[0, NP:NP + N_NODES, :] + n_pair[1, NP:NP + N_NODES, :]
    ssum_p = s0[:N_NODES] + s1[:N_NODES]
    ssum_n = s0[NP:NP + N_NODES] + s1[NP:NP + N_NODES]
    num = bp[:, None] * nsum_p + bn[:, None] * nsum_n
    den = bp * ssum_p + bn * ssum_n + 1e-9
    return num / den[:, None]


def _combine(nuu_ref, suu0_ref, suu1_ref, nui_ref, sui0_ref, sui1_ref,
             niu_ref, siu0_ref, siu1_ref, eep_ref):
    eep = eep_ref[...]
    mu = 0.5 * (_agg(nuu_ref[...], suu0_ref[...], suu1_ref[...], eep[1])
                + _agg(niu_ref[...], siu0_ref[...], siu1_ref[...], eep[5]))
    mi = _agg(nui_ref[...], sui0_ref[...], sui1_ref[...], eep[3])
    return mu, mi


def _combine_elu_body(nuu_ref, suu0_ref, suu1_ref, nui_ref, sui0_ref, sui1_ref,
                      niu_ref, siu0_ref, siu1_ref, eep_ref,
                      xu_ref, xi_ref):
    mu, mi = _combine(nuu_ref, suu0_ref, suu1_ref, nui_ref, sui0_ref, sui1_ref,
                      niu_ref, siu0_ref, siu1_ref, eep_ref)
    xu_ref[...] = jnp.where(mu > 0, mu, jnp.exp(jnp.minimum(mu, 0.0)) - 1.0)
    xi_ref[...] = jnp.where(mi > 0, mi, jnp.exp(jnp.minimum(mi, 0.0)) - 1.0)


def _final_body(nuu_ref, suu0_ref, suu1_ref, nui_ref, sui0_ref, sui1_ref,
                niu_ref, siu0_ref, siu1_ref, eep_ref,
                wc_ref, bc_ref, out_ref):
    mu, mi = _combine(nuu_ref, suu0_ref, suu1_ref, nui_ref, sui0_ref, sui1_ref,
                      niu_ref, siu0_ref, siu1_ref, eep_ref)
    hg = jnp.mean(mu, axis=0) + jnp.mean(mi, axis=0)
    val = jnp.sum(hg * wc_ref[...][:, 0]) + bc_ref[...][0]
    out_ref[...] = jax.nn.sigmoid(val).reshape(1)


_vtab_shapes = (
    jax.ShapeDtypeStruct((NP2, DIM), jnp.float32),
    jax.ShapeDtypeStruct((NP2, DIM), jnp.float32),
    jax.ShapeDtypeStruct((NP2, DIM), jnp.float32),
    jax.ShapeDtypeStruct((8, NP), jnp.float32),
)

_dense0 = pl.pallas_call(_dense0_body, out_shape=_vtab_shapes)

_combine_elu = pl.pallas_call(
    _combine_elu_body,
    out_shape=(
        jax.ShapeDtypeStruct((N_NODES, DIM), jnp.float32),
        jax.ShapeDtypeStruct((N_NODES, DIM), jnp.float32),
    ),
)

_final = pl.pallas_call(
    _final_body,
    out_shape=jax.ShapeDtypeStruct((1,), jnp.float32),
)


_sc_mesh = plsc.VectorSubcoreMesh(core_axis_name="c", subcore_axis_name="s")

import dataclasses as _dataclasses

_sc_params = pltpu.CompilerParams()
if "needs_layout_passes" in pltpu.CompilerParams.__dataclass_fields__:
    _sc_params = _dataclasses.replace(_sc_params, needs_layout_passes=False)


@functools.partial(
    pl.kernel,
    out_type=tuple(
        [jax.ShapeDtypeStruct((N_TILES, NCH, CH), jnp.int32)] * 2 * 3
        + [jax.ShapeDtypeStruct((N_TILES, NCH, CH), jnp.float32)] * 3,
    ),
    mesh=_sc_mesh,
    compiler_params=_sc_params,
    scratch_types=[
        pltpu.VMEM((NCH, CH), jnp.int32),
        pltpu.VMEM((NCH, CH), jnp.int32),
        pltpu.VMEM((NP,), jnp.float32),
        pltpu.VMEM((NP,), jnp.float32),
        pltpu.VMEM((NCH, CH), jnp.float32),
    ],
)
def _route_edges(es0, ed0, es1, ed1, es2, ed2, src0, dst0, src1, dst1,
                 src2, dst2,
                 g0, g1, g2, x0, x1, x2, v0, v1, v2,
                 src_v, dst_v, es_v, ed_v, sval_v):
    cid = lax.axis_index("c")
    sid = lax.axis_index("s")
    wid = sid * 2 + cid

    for es_hbm, ed_hbm, src_hbm, dst_hbm, gidx_hbm, sidx_hbm, sval_hbm in (
            (es0, ed0, src0, dst0, g0, x0, v0),
            (es1, ed1, src1, dst1, g1, x1, v1),
            (es2, ed2, src2, dst2, g2, x2, v2),
    ):
        pltpu.sync_copy(src_hbm.at[wid], src_v)
        pltpu.sync_copy(dst_hbm.at[wid], dst_v)
        pltpu.sync_copy(es_hbm, es_v)
        pltpu.sync_copy(ed_hbm, ed_v)

        @pl.loop(0, NCH)
        def _(c):
            @pl.loop(0, CH, step=16)
            def _(j):
                s16 = src_v[c, pl.ds(j, 16)]
                d16 = dst_v[c, pl.ds(j, 16)]
                ev = plsc.load_gather(es_v, [s16])
                edv = plsc.load_gather(ed_v, [d16])
                neg = (ev + edv) < 0.0
                offs = jnp.where(neg, NP, 0).astype(jnp.int32)
                src_v[c, pl.ds(j, 16)] = s16 + offs
                dst_v[c, pl.ds(j, 16)] = d16 + offs
                sval_v[c, pl.ds(j, 16)] = jnp.exp(
                    jnp.where(neg, 0.2, 1.0) * ev)

        pltpu.sync_copy(src_v, gidx_hbm.at[wid])
        pltpu.sync_copy(dst_v, sidx_hbm.at[wid])
        pltpu.sync_copy(sval_v, sval_hbm.at[wid])


@functools.partial(
    pl.kernel,
    out_type=(
        jax.ShapeDtypeStruct((2, NP2, DIM), jnp.float32),
        jax.ShapeDtypeStruct((NP2,), jnp.float32),
        jax.ShapeDtypeStruct((NP2,), jnp.float32),
    ),
    mesh=_sc_mesh,
    compiler_params=_sc_params,
    scratch_types=[
        pltpu.VMEM((NCH, CH), jnp.int32),
        pltpu.VMEM((NCH, CH), jnp.int32),
        pltpu.VMEM((NCH, CH), jnp.float32),
        pltpu.VMEM((NBUF, CH, DIM), jnp.float32),
        pltpu.VMEM_SHARED((NP2, DIM), jnp.float32),
        pltpu.VMEM_SHARED((NP2,), jnp.float32),
    ] + [pltpu.SemaphoreType.DMA] * (2 * NBUF),
)
def _rel_edges(vtab_hbm, gidx_hbm, sidx_hbm, sval_hbm, zn_hbm, zs_hbm,
               n_out, s0_out, s1_out,
               gidx_v, sidx_v, sval_v, bufs, acc, s_acc, *sems):
    gsems = sems[:NBUF]
    ssems = sems[NBUF:]
    cid = lax.axis_index("c")
    sid = lax.axis_index("s")
    wid = sid * 2 + cid

    pltpu.sync_copy(gidx_hbm.at[wid], gidx_v)
    pltpu.sync_copy(sidx_hbm.at[wid], sidx_v)
    pltpu.sync_copy(sval_hbm.at[wid], sval_v)

    base2 = sid * ROWS2_PER_TILE
    pltpu.sync_copy(zn_hbm.at[pl.ds(base2, ROWS2_PER_TILE)],
                    acc.at[pl.ds(base2, ROWS2_PER_TILE)])

    @pl.when(sid == 0)
    def _():
        pltpu.sync_copy(zs_hbm, s_acc)

    plsc.subcore_barrier()

    def _wait_gather(b):
        pltpu.make_async_copy(vtab_hbm.at[gidx_v.at[0]], bufs.at[b],
                              gsems[b]).wait()

    def _wait_scatter(b):
        pltpu.make_async_copy(bufs.at[b], acc.at[sidx_v.at[0]], ssems[b]).wait()
        pltpu.make_async_copy(sval_v.at[0], s_acc.at[sidx_v.at[0]],
                              ssems[b]).wait()

    for b in range(NBUF):
        pltpu.async_copy(vtab_hbm.at[gidx_v.at[b]], bufs.at[b], gsems[b])

    @pl.loop(0, NCH, step=NBUF)
    def _(c0):
        for b in range(NBUF):
            cc = c0 + b
            _wait_gather(b)
            pltpu.async_copy(bufs.at[b], acc.at[sidx_v.at[cc]], ssems[b],
                             add=True)
            pltpu.async_copy(sval_v.at[cc], s_acc.at[sidx_v.at[cc]],
                             ssems[b], add=True)

            @pl.when(cc + NBUF < NCH)
            def _():
                _wait_scatter(b)
                pltpu.async_copy(vtab_hbm.at[gidx_v.at[cc + NBUF]], bufs.at[b],
                                 gsems[b])

    for b in range(NBUF):
        _wait_scatter(b)

    plsc.subcore_barrier()

    pltpu.sync_copy(acc.at[pl.ds(base2, ROWS2_PER_TILE)],
                    n_out.at[cid, pl.ds(base2, ROWS2_PER_TILE)])

    @pl.when((sid == 0) & (cid == 0))
    def _():
        pltpu.sync_copy(s_acc, s0_out)

    @pl.when((sid == 0) & (cid == 1))
    def _():
        pltpu.sync_copy(s_acc, s1_out)


def _prep_edges(ei):
    pad = EPAD - E
    src = jnp.concatenate(
        [ei[0], (jnp.arange(pad, dtype=jnp.int32) % N_NODES)])
    dst = jnp.concatenate(
        [ei[1], N_NODES + (jnp.arange(pad, dtype=jnp.int32) % 8)])
    return src.reshape(N_TILES, NCH, CH), dst.reshape(N_TILES, NCH, CH)


def kernel(x_user, x_item, edge_uu, edge_ui, edge_iu,
           W_0_uu, as_0_uu, ad_0_uu, W_0_ui, as_0_ui, ad_0_ui,
           W_0_iu, as_0_iu, ad_0_iu, W_1_uu, as_1_uu, ad_1_uu,
           W_1_ui, as_1_ui, ad_1_ui, W_1_iu, as_1_iu, ad_1_iu,
           Wc, bc):
    suu, duu = _prep_edges(edge_uu)
    sui, dui = _prep_edges(edge_ui)
    siu, diu = _prep_edges(edge_iu)

    a0 = jnp.stack([as_0_uu, ad_0_uu, as_0_ui, ad_0_ui, as_0_iu, ad_0_iu])
    a1 = jnp.stack([as_1_uu, ad_1_uu, as_1_ui, ad_1_ui, as_1_iu, ad_1_iu])

    zn = jnp.zeros((NP2, DIM), jnp.float32)
    zs = jnp.zeros((NP2,), jnp.float32)

    vuu, vui, viu, ee0 = _dense0(x_user, x_item, W_0_uu, W_0_ui, W_0_iu, a0)
    guu, gui, giu, xuu, xui, xiu, wuu, wui, wiu = _route_edges(
        ee0[0], ee0[1], ee0[2], ee0[3], ee0[4], ee0[5],
        suu, duu, sui, dui, siu, diu)
    nuu, suu0, suu1 = _rel_edges(vuu, guu, xuu, wuu, zn, zs)
    nui, sui0, sui1 = _rel_edges(vui, gui, xui, wui, zn, zs)
    niu, siu0, siu1 = _rel_edges(viu, giu, xiu, wiu, zn, zs)

    xu1, xi1 = _combine_elu(
        nuu, suu0, suu1, nui, sui0, sui1, niu, siu0, siu1, ee0)
    vuu, vui, viu, ee1 = _dense0(xu1, xi1, W_1_uu, W_1_ui, W_1_iu, a1)
    guu, gui, giu, xuu, xui, xiu, wuu, wui, wiu = _route_edges(
        ee1[0], ee1[1], ee1[2], ee1[3], ee1[4], ee1[5],
        suu, duu, sui, dui, siu, diu)
    nuu, suu0, suu1 = _rel_edges(vuu, guu, xuu, wuu, zn, zs)
    nui, sui0, sui1 = _rel_edges(vui, gui, xui, wui, zn, zs)
    niu, siu0, siu1 = _rel_edges(viu, giu, xiu, wiu, zn, zs)

    return _final(nuu, suu0, suu1, nui, sui0, sui1, niu, siu0, siu1, ee1,
                  Wc, bc)


# speedup vs baseline: 1.1702x; 1.1278x over previous
import functools

import jax
import jax.numpy as jnp
from jax import lax
from jax.experimental import pallas as pl
from jax.experimental.pallas import tpu as pltpu
from jax.experimental.pallas import tpu_sc as plsc

N_NODES = 5000
DIM = 128
E = 160000
NP = 5120
NP2 = 2 * NP
ROWS2_PER_TILE = NP2 // 16
NBUF = 2
N_TILES = 32
TE = 5120
EPAD = N_TILES * TE
CH = 64
NCH = TE // CH

_HI = jax.lax.Precision.HIGHEST


def _dot(a, b):
    return jnp.dot(a, b, precision=_HI, preferred_element_type=jnp.float32)


def _make_vtab(h, es):
    pad = jnp.zeros((NP - N_NODES, DIM), jnp.float32)
    return jnp.concatenate([
        jnp.exp(es)[:, None] * h, pad,
        jnp.exp(0.2 * es)[:, None] * h, pad,
    ], axis=0).astype(jnp.bfloat16)


def _dense0_body(xu_ref, xi_ref, wuu_ref, wui_ref, wiu_ref, a_ref,
                 vuu_ref, vui_ref, viu_ref, ee_ref):
    xu = xu_ref[...]
    xi = xi_ref[...]
    huu = _dot(xu, wuu_ref[...])
    hui = _dot(xu, wui_ref[...])
    hiu = _dot(xi, wiu_ref[...])
    hdui = _dot(xi, wui_ref[...])
    hdiu = _dot(xu, wiu_ref[...])
    a = a_ref[...]
    mv = lambda h, v: jnp.sum(h * v[None, :], axis=1)
    es_uu = mv(huu, a[0])
    es_ui = mv(hui, a[2])
    es_iu = mv(hiu, a[4])
    vuu_ref[...] = _make_vtab(huu, es_uu)
    vui_ref[...] = _make_vtab(hui, es_ui)
    viu_ref[...] = _make_vtab(hiu, es_iu)
    ee = jnp.stack([
        es_uu, mv(huu, a[1]),
        es_ui, mv(hdui, a[3]),
        es_iu, mv(hdiu, a[5]),
        jnp.zeros((N_NODES,), jnp.float32),
        jnp.zeros((N_NODES,), jnp.float32),
    ])
    ee_ref[...] = jnp.concatenate(
        [ee, jnp.zeros((8, NP - N_NODES), jnp.float32)], axis=1)


def _agg(n_pair, s0, s1, ed):
    bp = jnp.exp(ed[:N_NODES])
    bn = jnp.exp(0.2 * ed[:N_NODES])
    nf = n_pair.astype(jnp.float32)
    nsum_p = nf[0, :N_NODES, :] + nf[1, :N_NODES, :]
    nsum_n = nf[0, NP:NP + N_NODES, :] + nf[1, NP:NP + N_NODES, :]
    ssum_p = s0[:N_NODES] + s1[:N_NODES]
    ssum_n = s0[NP:NP + N_NODES] + s1[NP:NP + N_NODES]
    num = bp[:, None] * nsum_p + bn[:, None] * nsum_n
    den = bp * ssum_p + bn * ssum_n + 1e-9
    return num / den[:, None]


def _combine(nuu_ref, suu0_ref, suu1_ref, nui_ref, sui0_ref, sui1_ref,
             niu_ref, siu0_ref, siu1_ref, eep_ref):
    eep = eep_ref[...]
    mu = 0.5 * (_agg(nuu_ref[...], suu0_ref[...], suu1_ref[...], eep[1])
                + _agg(niu_ref[...], siu0_ref[...], siu1_ref[...], eep[5]))
    mi = _agg(nui_ref[...], sui0_ref[...], sui1_ref[...], eep[3])
    return mu, mi


def _combine_elu_body(nuu_ref, suu0_ref, suu1_ref, nui_ref, sui0_ref, sui1_ref,
                      niu_ref, siu0_ref, siu1_ref, eep_ref,
                      xu_ref, xi_ref):
    mu, mi = _combine(nuu_ref, suu0_ref, suu1_ref, nui_ref, sui0_ref, sui1_ref,
                      niu_ref, siu0_ref, siu1_ref, eep_ref)
    xu_ref[...] = jnp.where(mu > 0, mu, jnp.exp(jnp.minimum(mu, 0.0)) - 1.0)
    xi_ref[...] = jnp.where(mi > 0, mi, jnp.exp(jnp.minimum(mi, 0.0)) - 1.0)


def _final_body(nuu_ref, suu0_ref, suu1_ref, nui_ref, sui0_ref, sui1_ref,
                niu_ref, siu0_ref, siu1_ref, eep_ref,
                wc_ref, bc_ref, out_ref):
    mu, mi = _combine(nuu_ref, suu0_ref, suu1_ref, nui_ref, sui0_ref, sui1_ref,
                      niu_ref, siu0_ref, siu1_ref, eep_ref)
    hg = jnp.mean(mu, axis=0) + jnp.mean(mi, axis=0)
    val = jnp.sum(hg * wc_ref[...][:, 0]) + bc_ref[...][0]
    out_ref[...] = jax.nn.sigmoid(val).reshape(1)


_vtab_shapes = (
    jax.ShapeDtypeStruct((NP2, DIM), jnp.bfloat16),
    jax.ShapeDtypeStruct((NP2, DIM), jnp.bfloat16),
    jax.ShapeDtypeStruct((NP2, DIM), jnp.bfloat16),
    jax.ShapeDtypeStruct((8, NP), jnp.float32),
)

_dense0 = pl.pallas_call(_dense0_body, out_shape=_vtab_shapes)

_combine_elu = pl.pallas_call(
    _combine_elu_body,
    out_shape=(
        jax.ShapeDtypeStruct((N_NODES, DIM), jnp.float32),
        jax.ShapeDtypeStruct((N_NODES, DIM), jnp.float32),
    ),
)

_final = pl.pallas_call(
    _final_body,
    out_shape=jax.ShapeDtypeStruct((1,), jnp.float32),
)


_sc_mesh = plsc.VectorSubcoreMesh(core_axis_name="c", subcore_axis_name="s")

import dataclasses as _dataclasses

_sc_params = pltpu.CompilerParams()
if "needs_layout_passes" in pltpu.CompilerParams.__dataclass_fields__:
    _sc_params = _dataclasses.replace(_sc_params, needs_layout_passes=False)
_sc_pump_params = _dataclasses.replace(_sc_params, use_tc_tiling_on_sc=False)


@functools.partial(
    pl.kernel,
    out_type=tuple(
        [jax.ShapeDtypeStruct((N_TILES, NCH, CH), jnp.int32)] * 2 * 3
        + [jax.ShapeDtypeStruct((N_TILES, NCH, CH), jnp.float32)] * 3,
    ),
    mesh=_sc_mesh,
    compiler_params=_sc_params,
    scratch_types=[
        pltpu.VMEM((NCH, CH), jnp.int32),
        pltpu.VMEM((NCH, CH), jnp.int32),
        pltpu.VMEM((NP,), jnp.float32),
        pltpu.VMEM((NP,), jnp.float32),
        pltpu.VMEM((NCH, CH), jnp.float32),
    ],
)
def _route_edges(es0, ed0, es1, ed1, es2, ed2, src0, dst0, src1, dst1,
                 src2, dst2,
                 g0, g1, g2, x0, x1, x2, v0, v1, v2,
                 src_v, dst_v, es_v, ed_v, sval_v):
    cid = lax.axis_index("c")
    sid = lax.axis_index("s")
    wid = sid * 2 + cid

    for es_hbm, ed_hbm, src_hbm, dst_hbm, gidx_hbm, sidx_hbm, sval_hbm in (
            (es0, ed0, src0, dst0, g0, x0, v0),
            (es1, ed1, src1, dst1, g1, x1, v1),
            (es2, ed2, src2, dst2, g2, x2, v2),
    ):
        pltpu.sync_copy(src_hbm.at[wid], src_v)
        pltpu.sync_copy(dst_hbm.at[wid], dst_v)
        pltpu.sync_copy(es_hbm, es_v)
        pltpu.sync_copy(ed_hbm, ed_v)

        @pl.loop(0, NCH)
        def _(c):
            @pl.loop(0, CH, step=16)
            def _(j):
                s16 = src_v[c, pl.ds(j, 16)]
                d16 = dst_v[c, pl.ds(j, 16)]
                ev = plsc.load_gather(es_v, [s16])
                edv = plsc.load_gather(ed_v, [d16])
                neg = (ev + edv) < 0.0
                offs = jnp.where(neg, NP, 0).astype(jnp.int32)
                src_v[c, pl.ds(j, 16)] = s16 + offs
                dst_v[c, pl.ds(j, 16)] = d16 + offs
                sval_v[c, pl.ds(j, 16)] = jnp.exp(
                    jnp.where(neg, 0.2, 1.0) * ev)

        pltpu.sync_copy(src_v, gidx_hbm.at[wid])
        pltpu.sync_copy(dst_v, sidx_hbm.at[wid])
        pltpu.sync_copy(sval_v, sval_hbm.at[wid])


@functools.partial(
    pl.kernel,
    out_type=(
        jax.ShapeDtypeStruct((2, NP2, DIM), jnp.bfloat16),
        jax.ShapeDtypeStruct((NP2,), jnp.float32),
        jax.ShapeDtypeStruct((NP2,), jnp.float32),
    ),
    mesh=_sc_mesh,
    compiler_params=_sc_pump_params,
    scratch_types=[
        pltpu.VMEM((NCH, CH), jnp.int32),
        pltpu.VMEM((NCH, CH), jnp.int32),
        pltpu.VMEM((NCH, CH), jnp.float32),
        pltpu.VMEM((NBUF, CH, DIM), jnp.bfloat16),
        pltpu.VMEM_SHARED((NP2, DIM), jnp.bfloat16),
        pltpu.VMEM_SHARED((NP2,), jnp.float32),
    ] + [pltpu.SemaphoreType.DMA] * (2 * NBUF),
)
def _rel_edges(vtab_hbm, gidx_hbm, sidx_hbm, sval_hbm, zn_hbm, zs_hbm,
               n_out, s0_out, s1_out,
               gidx_v, sidx_v, sval_v, bufs, acc, s_acc, *sems):
    gsems = sems[:NBUF]
    ssems = sems[NBUF:]
    cid = lax.axis_index("c")
    sid = lax.axis_index("s")
    wid = sid * 2 + cid

    pltpu.sync_copy(gidx_hbm.at[wid], gidx_v)
    pltpu.sync_copy(sidx_hbm.at[wid], sidx_v)
    pltpu.sync_copy(sval_hbm.at[wid], sval_v)

    base2 = sid * ROWS2_PER_TILE
    pltpu.sync_copy(zn_hbm.at[pl.ds(base2, ROWS2_PER_TILE)],
                    acc.at[pl.ds(base2, ROWS2_PER_TILE)])

    @pl.when(sid == 0)
    def _():
        pltpu.sync_copy(zs_hbm, s_acc)

    plsc.subcore_barrier()

    def _wait_gather(b):
        pltpu.make_async_copy(vtab_hbm.at[gidx_v.at[0]], bufs.at[b],
                              gsems[b]).wait()

    def _wait_scatter(b):
        pltpu.make_async_copy(bufs.at[b], acc.at[sidx_v.at[0]], ssems[b]).wait()
        pltpu.make_async_copy(sval_v.at[0], s_acc.at[sidx_v.at[0]],
                              ssems[b]).wait()

    for b in range(NBUF):
        pltpu.async_copy(vtab_hbm.at[gidx_v.at[b]], bufs.at[b], gsems[b])

    @pl.loop(0, NCH, step=NBUF)
    def _(c0):
        for b in range(NBUF):
            cc = c0 + b
            _wait_gather(b)
            pltpu.async_copy(bufs.at[b], acc.at[sidx_v.at[cc]], ssems[b],
                             add=True)
            pltpu.async_copy(sval_v.at[cc], s_acc.at[sidx_v.at[cc]],
                             ssems[b], add=True)

            @pl.when(cc + NBUF < NCH)
            def _():
                _wait_scatter(b)
                pltpu.async_copy(vtab_hbm.at[gidx_v.at[cc + NBUF]], bufs.at[b],
                                 gsems[b])

    for b in range(NBUF):
        _wait_scatter(b)

    plsc.subcore_barrier()

    pltpu.sync_copy(acc.at[pl.ds(base2, ROWS2_PER_TILE)],
                    n_out.at[cid, pl.ds(base2, ROWS2_PER_TILE)])

    @pl.when((sid == 0) & (cid == 0))
    def _():
        pltpu.sync_copy(s_acc, s0_out)

    @pl.when((sid == 0) & (cid == 1))
    def _():
        pltpu.sync_copy(s_acc, s1_out)


def _prep_edges(ei):
    pad = EPAD - E
    src = jnp.concatenate(
        [ei[0], (jnp.arange(pad, dtype=jnp.int32) % N_NODES)])
    dst = jnp.concatenate(
        [ei[1], N_NODES + (jnp.arange(pad, dtype=jnp.int32) % 8)])
    return src.reshape(N_TILES, NCH, CH), dst.reshape(N_TILES, NCH, CH)


def kernel(x_user, x_item, edge_uu, edge_ui, edge_iu,
           W_0_uu, as_0_uu, ad_0_uu, W_0_ui, as_0_ui, ad_0_ui,
           W_0_iu, as_0_iu, ad_0_iu, W_1_uu, as_1_uu, ad_1_uu,
           W_1_ui, as_1_ui, ad_1_ui, W_1_iu, as_1_iu, ad_1_iu,
           Wc, bc):
    suu, duu = _prep_edges(edge_uu)
    sui, dui = _prep_edges(edge_ui)
    siu, diu = _prep_edges(edge_iu)

    a0 = jnp.stack([as_0_uu, ad_0_uu, as_0_ui, ad_0_ui, as_0_iu, ad_0_iu])
    a1 = jnp.stack([as_1_uu, ad_1_uu, as_1_ui, ad_1_ui, as_1_iu, ad_1_iu])

    zn = jnp.zeros((NP2, DIM), jnp.bfloat16)
    zs = jnp.zeros((NP2,), jnp.float32)

    vuu, vui, viu, ee0 = _dense0(x_user, x_item, W_0_uu, W_0_ui, W_0_iu, a0)
    guu, gui, giu, xuu, xui, xiu, wuu, wui, wiu = _route_edges(
        ee0[0], ee0[1], ee0[2], ee0[3], ee0[4], ee0[5],
        suu, duu, sui, dui, siu, diu)
    nuu, suu0, suu1 = _rel_edges(vuu, guu, xuu, wuu, zn, zs)
    nui, sui0, sui1 = _rel_edges(vui, gui, xui, wui, zn, zs)
    niu, siu0, siu1 = _rel_edges(viu, giu, xiu, wiu, zn, zs)

    xu1, xi1 = _combine_elu(
        nuu, suu0, suu1, nui, sui0, sui1, niu, siu0, siu1, ee0)
    vuu, vui, viu, ee1 = _dense0(xu1, xi1, W_1_uu, W_1_ui, W_1_iu, a1)
    guu, gui, giu, xuu, xui, xiu, wuu, wui, wiu = _route_edges(
        ee1[0], ee1[1], ee1[2], ee1[3], ee1[4], ee1[5],
        suu, duu, sui, dui, siu, diu)
    nuu, suu0, suu1 = _rel_edges(vuu, guu, xuu, wuu, zn, zs)
    nui, sui0, sui1 = _rel_edges(vui, gui, xui, wui, zn, zs)
    niu, siu0, siu1 = _rel_edges(viu, giu, xiu, wiu, zn, zs)

    return _final(nuu, suu0, suu1, nui, sui0, sui1, niu, siu0, siu1, ee1,
                  Wc, bc)


# speedup vs baseline: 1.5004x; 1.2822x over previous
import functools

import jax
import jax.numpy as jnp
from jax import lax
from jax.experimental import pallas as pl
from jax.experimental.pallas import tpu as pltpu
from jax.experimental.pallas import tpu_sc as plsc

N_NODES = 5000
DIM = 128
E = 160000
NP = 5120
NP2 = 2 * NP
ROWS2_PER_TILE = NP2 // 16
NBUF = 4
N_TILES = 32
TE = 5120
EPAD = N_TILES * TE
CH = 64
NCH = TE // CH

_HI = jax.lax.Precision.HIGHEST


def _dot(a, b):
    return jnp.dot(a, b, precision=_HI, preferred_element_type=jnp.float32)


def _make_vtab(h, es):
    pad = jnp.zeros((NP - N_NODES, DIM), jnp.float32)
    return jnp.concatenate([
        jnp.exp(es)[:, None] * h, pad,
        jnp.exp(0.2 * es)[:, None] * h, pad,
    ], axis=0).astype(jnp.bfloat16)


def _dense0_body(xu_ref, xi_ref, wuu_ref, wui_ref, wiu_ref, a_ref,
                 vuu_ref, vui_ref, viu_ref, ee_ref):
    xu = xu_ref[...]
    xi = xi_ref[...]
    huu = _dot(xu, wuu_ref[...])
    hui = _dot(xu, wui_ref[...])
    hiu = _dot(xi, wiu_ref[...])
    hdui = _dot(xi, wui_ref[...])
    hdiu = _dot(xu, wiu_ref[...])
    a = a_ref[...]
    mv = lambda h, v: jnp.sum(h * v[None, :], axis=1)
    es_uu = mv(huu, a[0])
    es_ui = mv(hui, a[2])
    es_iu = mv(hiu, a[4])
    vuu_ref[...] = _make_vtab(huu, es_uu)
    vui_ref[...] = _make_vtab(hui, es_ui)
    viu_ref[...] = _make_vtab(hiu, es_iu)
    ee = jnp.stack([
        es_uu, mv(huu, a[1]),
        es_ui, mv(hdui, a[3]),
        es_iu, mv(hdiu, a[5]),
        jnp.zeros((N_NODES,), jnp.float32),
        jnp.zeros((N_NODES,), jnp.float32),
    ])
    ee_ref[...] = jnp.concatenate(
        [ee, jnp.zeros((8, NP - N_NODES), jnp.float32)], axis=1)


def _agg(n_pair, s0, s1, ed):
    bp = jnp.exp(ed[:N_NODES])
    bn = jnp.exp(0.2 * ed[:N_NODES])
    nf = n_pair.astype(jnp.float32)
    nsum_p = nf[0, :N_NODES, :] + nf[1, :N_NODES, :]
    nsum_n = nf[0, NP:NP + N_NODES, :] + nf[1, NP:NP + N_NODES, :]
    ssum_p = s0[:N_NODES] + s1[:N_NODES]
    ssum_n = s0[NP:NP + N_NODES] + s1[NP:NP + N_NODES]
    num = bp[:, None] * nsum_p + bn[:, None] * nsum_n
    den = bp * ssum_p + bn * ssum_n + 1e-9
    return num / den[:, None]


def _combine(nuu_ref, suu0_ref, suu1_ref, nui_ref, sui0_ref, sui1_ref,
             niu_ref, siu0_ref, siu1_ref, eep_ref):
    eep = eep_ref[...]
    mu = 0.5 * (_agg(nuu_ref[...], suu0_ref[...], suu1_ref[...], eep[1])
                + _agg(niu_ref[...], siu0_ref[...], siu1_ref[...], eep[5]))
    mi = _agg(nui_ref[...], sui0_ref[...], sui1_ref[...], eep[3])
    return mu, mi


def _combine_elu_body(nuu_ref, suu0_ref, suu1_ref, nui_ref, sui0_ref, sui1_ref,
                      niu_ref, siu0_ref, siu1_ref, eep_ref,
                      xu_ref, xi_ref):
    mu, mi = _combine(nuu_ref, suu0_ref, suu1_ref, nui_ref, sui0_ref, sui1_ref,
                      niu_ref, siu0_ref, siu1_ref, eep_ref)
    xu_ref[...] = jnp.where(mu > 0, mu, jnp.exp(jnp.minimum(mu, 0.0)) - 1.0)
    xi_ref[...] = jnp.where(mi > 0, mi, jnp.exp(jnp.minimum(mi, 0.0)) - 1.0)


def _final_body(nuu_ref, suu0_ref, suu1_ref, nui_ref, sui0_ref, sui1_ref,
                niu_ref, siu0_ref, siu1_ref, eep_ref,
                wc_ref, bc_ref, out_ref):
    mu, mi = _combine(nuu_ref, suu0_ref, suu1_ref, nui_ref, sui0_ref, sui1_ref,
                      niu_ref, siu0_ref, siu1_ref, eep_ref)
    hg = jnp.mean(mu, axis=0) + jnp.mean(mi, axis=0)
    val = jnp.sum(hg * wc_ref[...][:, 0]) + bc_ref[...][0]
    out_ref[...] = jax.nn.sigmoid(val).reshape(1)


_vtab_shapes = (
    jax.ShapeDtypeStruct((NP2, DIM), jnp.bfloat16),
    jax.ShapeDtypeStruct((NP2, DIM), jnp.bfloat16),
    jax.ShapeDtypeStruct((NP2, DIM), jnp.bfloat16),
    jax.ShapeDtypeStruct((8, NP), jnp.float32),
)

_dense0 = pl.pallas_call(_dense0_body, out_shape=_vtab_shapes)

_combine_elu = pl.pallas_call(
    _combine_elu_body,
    out_shape=(
        jax.ShapeDtypeStruct((N_NODES, DIM), jnp.float32),
        jax.ShapeDtypeStruct((N_NODES, DIM), jnp.float32),
    ),
)

_final = pl.pallas_call(
    _final_body,
    out_shape=jax.ShapeDtypeStruct((1,), jnp.float32),
)


_sc_mesh = plsc.VectorSubcoreMesh(core_axis_name="c", subcore_axis_name="s")

import dataclasses as _dataclasses

_sc_params = pltpu.CompilerParams()
if "needs_layout_passes" in pltpu.CompilerParams.__dataclass_fields__:
    _sc_params = _dataclasses.replace(_sc_params, needs_layout_passes=False)
_sc_pump_params = _dataclasses.replace(_sc_params, use_tc_tiling_on_sc=False)


@functools.partial(
    pl.kernel,
    out_type=(
        jax.ShapeDtypeStruct((2, NP2, DIM), jnp.bfloat16),
        jax.ShapeDtypeStruct((NP2,), jnp.float32),
        jax.ShapeDtypeStruct((NP2,), jnp.float32),
    ),
    mesh=_sc_mesh,
    compiler_params=_sc_pump_params,
    scratch_types=[
        pltpu.VMEM((NCH, CH), jnp.int32),
        pltpu.VMEM((NCH, CH), jnp.int32),
        pltpu.VMEM((NP,), jnp.float32),
        pltpu.VMEM((NP,), jnp.float32),
        pltpu.VMEM((NCH, CH), jnp.float32),
        pltpu.VMEM((NBUF, CH, DIM), jnp.bfloat16),
        pltpu.VMEM_SHARED((NP2, DIM), jnp.bfloat16),
        pltpu.VMEM_SHARED((NP2,), jnp.float32),
    ] + [pltpu.SemaphoreType.DMA] * (2 * NBUF),
)
def _rel_edges(vtab_hbm, es_hbm, ed_hbm, src_hbm, dst_hbm, zn_hbm, zs_hbm,
               n_out, s0_out, s1_out,
               gidx_v, sidx_v, es_v, ed_v, sval_v, bufs, acc, s_acc, *sems):
    gsems = sems[:NBUF]
    ssems = sems[NBUF:]
    cid = lax.axis_index("c")
    sid = lax.axis_index("s")
    wid = sid * 2 + cid

    pltpu.sync_copy(src_hbm.at[wid], gidx_v)
    pltpu.sync_copy(dst_hbm.at[wid], sidx_v)
    pltpu.sync_copy(es_hbm, es_v)
    pltpu.sync_copy(ed_hbm, ed_v)

    base2 = sid * ROWS2_PER_TILE
    pltpu.sync_copy(zn_hbm.at[pl.ds(base2, ROWS2_PER_TILE)],
                    acc.at[pl.ds(base2, ROWS2_PER_TILE)])

    @pl.when(sid == 0)
    def _():
        pltpu.sync_copy(zs_hbm, s_acc)

    @pl.loop(0, NCH)
    def _(c):
        @pl.loop(0, CH, step=16)
        def _(j):
            s16 = gidx_v[c, pl.ds(j, 16)]
            d16 = sidx_v[c, pl.ds(j, 16)]
            ev = plsc.load_gather(es_v, [s16])
            edv = plsc.load_gather(ed_v, [d16])
            neg = (ev + edv) < 0.0
            offs = jnp.where(neg, NP, 0).astype(jnp.int32)
            gidx_v[c, pl.ds(j, 16)] = s16 + offs
            sidx_v[c, pl.ds(j, 16)] = d16 + offs
            sval_v[c, pl.ds(j, 16)] = jnp.exp(jnp.where(neg, 0.2, 1.0) * ev)

    plsc.subcore_barrier()

    def _wait_gather(b):
        pltpu.make_async_copy(vtab_hbm.at[gidx_v.at[0]], bufs.at[b],
                              gsems[b]).wait()

    def _wait_scatter(b):
        pltpu.make_async_copy(bufs.at[b], acc.at[sidx_v.at[0]], ssems[b]).wait()
        pltpu.make_async_copy(sval_v.at[0], s_acc.at[sidx_v.at[0]],
                              ssems[b]).wait()

    for b in range(NBUF):
        pltpu.async_copy(vtab_hbm.at[gidx_v.at[b]], bufs.at[b], gsems[b])

    @pl.loop(0, NCH, step=NBUF)
    def _(c0):
        for b in range(NBUF):
            cc = c0 + b
            _wait_gather(b)
            pltpu.async_copy(bufs.at[b], acc.at[sidx_v.at[cc]], ssems[b],
                             add=True)
            pltpu.async_copy(sval_v.at[cc], s_acc.at[sidx_v.at[cc]],
                             ssems[b], add=True)

            @pl.when(cc + NBUF < NCH)
            def _():
                _wait_scatter(b)
                pltpu.async_copy(vtab_hbm.at[gidx_v.at[cc + NBUF]], bufs.at[b],
                                 gsems[b])

    for b in range(NBUF):
        _wait_scatter(b)

    plsc.subcore_barrier()

    pltpu.sync_copy(acc.at[pl.ds(base2, ROWS2_PER_TILE)],
                    n_out.at[cid, pl.ds(base2, ROWS2_PER_TILE)])

    @pl.when((sid == 0) & (cid == 0))
    def _():
        pltpu.sync_copy(s_acc, s0_out)

    @pl.when((sid == 0) & (cid == 1))
    def _():
        pltpu.sync_copy(s_acc, s1_out)


def _prep_edges(ei):
    pad = EPAD - E
    src = jnp.concatenate(
        [ei[0], (jnp.arange(pad, dtype=jnp.int32) % N_NODES)])
    dst = jnp.concatenate(
        [ei[1], N_NODES + (jnp.arange(pad, dtype=jnp.int32) % 8)])
    return src.reshape(N_TILES, NCH, CH), dst.reshape(N_TILES, NCH, CH)


def kernel(x_user, x_item, edge_uu, edge_ui, edge_iu,
           W_0_uu, as_0_uu, ad_0_uu, W_0_ui, as_0_ui, ad_0_ui,
           W_0_iu, as_0_iu, ad_0_iu, W_1_uu, as_1_uu, ad_1_uu,
           W_1_ui, as_1_ui, ad_1_ui, W_1_iu, as_1_iu, ad_1_iu,
           Wc, bc):
    suu, duu = _prep_edges(edge_uu)
    sui, dui = _prep_edges(edge_ui)
    siu, diu = _prep_edges(edge_iu)

    a0 = jnp.stack([as_0_uu, ad_0_uu, as_0_ui, ad_0_ui, as_0_iu, ad_0_iu])
    a1 = jnp.stack([as_1_uu, ad_1_uu, as_1_ui, ad_1_ui, as_1_iu, ad_1_iu])

    zn = jnp.zeros((NP2, DIM), jnp.bfloat16)
    zs = jnp.zeros((NP2,), jnp.float32)

    vuu, vui, viu, ee0 = _dense0(x_user, x_item, W_0_uu, W_0_ui, W_0_iu, a0)
    nuu, suu0, suu1 = _rel_edges(vuu, ee0[0], ee0[1], suu, duu, zn, zs)
    nui, sui0, sui1 = _rel_edges(vui, ee0[2], ee0[3], sui, dui, zn, zs)
    niu, siu0, siu1 = _rel_edges(viu, ee0[4], ee0[5], siu, diu, zn, zs)

    xu1, xi1 = _combine_elu(
        nuu, suu0, suu1, nui, sui0, sui1, niu, siu0, siu1, ee0)
    vuu, vui, viu, ee1 = _dense0(xu1, xi1, W_1_uu, W_1_ui, W_1_iu, a1)
    nuu, suu0, suu1 = _rel_edges(vuu, ee1[0], ee1[1], suu, duu, zn, zs)
    nui, sui0, sui1 = _rel_edges(vui, ee1[2], ee1[3], sui, dui, zn, zs)
    niu, siu0, siu1 = _rel_edges(viu, ee1[4], ee1[5], siu, diu, zn, zs)

    return _final(nuu, suu0, suu1, nui, sui0, sui1, niu, siu0, siu1, ee1,
                  Wc, bc)


# speedup vs baseline: 1.6116x; 1.0741x over previous
import functools

import jax
import jax.numpy as jnp
from jax import lax
from jax.experimental import pallas as pl
from jax.experimental.pallas import tpu as pltpu
from jax.experimental.pallas import tpu_sc as plsc

N_NODES = 5000
DIM = 128
E = 160000
NP = 5120
NP2 = 2 * NP
ROWS2_PER_TILE = NP2 // 16
NBUF = 4
N_TILES = 32
TE = 5120
EPAD = N_TILES * TE
CH = 64
NCH = TE // CH

_HI = jax.lax.Precision.DEFAULT


def _dot(a, b):
    return jnp.dot(a, b, precision=_HI, preferred_element_type=jnp.float32)


def _make_vtab(h, es):
    pad = jnp.zeros((NP - N_NODES, DIM), jnp.float32)
    return jnp.concatenate([
        jnp.exp(es)[:, None] * h, pad,
        jnp.exp(0.2 * es)[:, None] * h, pad,
    ], axis=0).astype(jnp.bfloat16)


def _dense0_body(xu_ref, xi_ref, wuu_ref, wui_ref, wiu_ref, a_ref,
                 vuu_ref, vui_ref, viu_ref, ee_ref):
    xu = xu_ref[...]
    xi = xi_ref[...]
    huu = _dot(xu, wuu_ref[...])
    hui = _dot(xu, wui_ref[...])
    hiu = _dot(xi, wiu_ref[...])
    hdui = _dot(xi, wui_ref[...])
    hdiu = _dot(xu, wiu_ref[...])
    a = a_ref[...]
    mv = lambda h, v: jnp.sum(h * v[None, :], axis=1)
    es_uu = mv(huu, a[0])
    es_ui = mv(hui, a[2])
    es_iu = mv(hiu, a[4])
    vuu_ref[...] = _make_vtab(huu, es_uu)
    vui_ref[...] = _make_vtab(hui, es_ui)
    viu_ref[...] = _make_vtab(hiu, es_iu)
    ee = jnp.stack([
        es_uu, mv(huu, a[1]),
        es_ui, mv(hdui, a[3]),
        es_iu, mv(hdiu, a[5]),
        jnp.zeros((N_NODES,), jnp.float32),
        jnp.zeros((N_NODES,), jnp.float32),
    ])
    ee_ref[...] = jnp.concatenate(
        [ee, jnp.zeros((8, NP - N_NODES), jnp.float32)], axis=1)


def _agg(n_pair, s0, s1, ed):
    bp = jnp.exp(ed[:N_NODES])
    bn = jnp.exp(0.2 * ed[:N_NODES])
    nf = n_pair.astype(jnp.float32)
    nsum_p = nf[0, :N_NODES, :] + nf[1, :N_NODES, :]
    nsum_n = nf[0, NP:NP + N_NODES, :] + nf[1, NP:NP + N_NODES, :]
    ssum_p = s0[:N_NODES] + s1[:N_NODES]
    ssum_n = s0[NP:NP + N_NODES] + s1[NP:NP + N_NODES]
    num = bp[:, None] * nsum_p + bn[:, None] * nsum_n
    den = bp * ssum_p + bn * ssum_n + 1e-9
    return num / den[:, None]


def _combine(nuu_ref, suu0_ref, suu1_ref, nui_ref, sui0_ref, sui1_ref,
             niu_ref, siu0_ref, siu1_ref, eep_ref):
    eep = eep_ref[...]
    mu = 0.5 * (_agg(nuu_ref[...], suu0_ref[...], suu1_ref[...], eep[1])
                + _agg(niu_ref[...], siu0_ref[...], siu1_ref[...], eep[5]))
    mi = _agg(nui_ref[...], sui0_ref[...], sui1_ref[...], eep[3])
    return mu, mi


def _combine_elu_body(nuu_ref, suu0_ref, suu1_ref, nui_ref, sui0_ref, sui1_ref,
                      niu_ref, siu0_ref, siu1_ref, eep_ref,
                      xu_ref, xi_ref):
    mu, mi = _combine(nuu_ref, suu0_ref, suu1_ref, nui_ref, sui0_ref, sui1_ref,
                      niu_ref, siu0_ref, siu1_ref, eep_ref)
    xu_ref[...] = jnp.where(mu > 0, mu, jnp.exp(jnp.minimum(mu, 0.0)) - 1.0)
    xi_ref[...] = jnp.where(mi > 0, mi, jnp.exp(jnp.minimum(mi, 0.0)) - 1.0)


def _final_body(nuu_ref, suu0_ref, suu1_ref, nui_ref, sui0_ref, sui1_ref,
                niu_ref, siu0_ref, siu1_ref, eep_ref,
                wc_ref, bc_ref, out_ref):
    mu, mi = _combine(nuu_ref, suu0_ref, suu1_ref, nui_ref, sui0_ref, sui1_ref,
                      niu_ref, siu0_ref, siu1_ref, eep_ref)
    hg = jnp.mean(mu, axis=0) + jnp.mean(mi, axis=0)
    val = jnp.sum(hg * wc_ref[...][:, 0]) + bc_ref[...][0]
    out_ref[...] = jax.nn.sigmoid(val).reshape(1)


_vtab_shapes = (
    jax.ShapeDtypeStruct((NP2, DIM), jnp.bfloat16),
    jax.ShapeDtypeStruct((NP2, DIM), jnp.bfloat16),
    jax.ShapeDtypeStruct((NP2, DIM), jnp.bfloat16),
    jax.ShapeDtypeStruct((8, NP), jnp.float32),
)

_dense0 = pl.pallas_call(_dense0_body, out_shape=_vtab_shapes)

_combine_elu = pl.pallas_call(
    _combine_elu_body,
    out_shape=(
        jax.ShapeDtypeStruct((N_NODES, DIM), jnp.float32),
        jax.ShapeDtypeStruct((N_NODES, DIM), jnp.float32),
    ),
)

_final = pl.pallas_call(
    _final_body,
    out_shape=jax.ShapeDtypeStruct((1,), jnp.float32),
)


_sc_mesh = plsc.VectorSubcoreMesh(core_axis_name="c", subcore_axis_name="s")

import dataclasses as _dataclasses

_sc_params = pltpu.CompilerParams()
if "needs_layout_passes" in pltpu.CompilerParams.__dataclass_fields__:
    _sc_params = _dataclasses.replace(_sc_params, needs_layout_passes=False)
_sc_pump_params = _dataclasses.replace(_sc_params, use_tc_tiling_on_sc=False)


@functools.partial(
    pl.kernel,
    out_type=(
        jax.ShapeDtypeStruct((2, NP2, DIM), jnp.bfloat16),
        jax.ShapeDtypeStruct((NP2,), jnp.float32),
        jax.ShapeDtypeStruct((NP2,), jnp.float32),
    ),
    mesh=_sc_mesh,
    compiler_params=_sc_pump_params,
    scratch_types=[
        pltpu.VMEM((NCH, CH), jnp.int32),
        pltpu.VMEM((NCH, CH), jnp.int32),
        pltpu.VMEM((NP,), jnp.float32),
        pltpu.VMEM((NP,), jnp.float32),
        pltpu.VMEM((NCH, CH), jnp.float32),
        pltpu.VMEM((NBUF, CH, DIM), jnp.bfloat16),
        pltpu.VMEM_SHARED((NP2, DIM), jnp.bfloat16),
        pltpu.VMEM_SHARED((NP2,), jnp.float32),
    ] + [pltpu.SemaphoreType.DMA] * (2 * NBUF),
)
def _rel_edges(vtab_hbm, es_hbm, ed_hbm, src_hbm, dst_hbm, zn_hbm, zs_hbm,
               n_out, s0_out, s1_out,
               gidx_v, sidx_v, es_v, ed_v, sval_v, bufs, acc, s_acc, *sems):
    gsems = sems[:NBUF]
    ssems = sems[NBUF:]
    cid = lax.axis_index("c")
    sid = lax.axis_index("s")
    wid = sid * 2 + cid

    pltpu.sync_copy(src_hbm.at[wid], gidx_v)
    pltpu.sync_copy(dst_hbm.at[wid], sidx_v)
    pltpu.sync_copy(es_hbm, es_v)
    pltpu.sync_copy(ed_hbm, ed_v)

    base2 = sid * ROWS2_PER_TILE
    pltpu.sync_copy(zn_hbm.at[pl.ds(base2, ROWS2_PER_TILE)],
                    acc.at[pl.ds(base2, ROWS2_PER_TILE)])

    @pl.when(sid == 0)
    def _():
        pltpu.sync_copy(zs_hbm, s_acc)

    @pl.loop(0, NCH)
    def _(c):
        @pl.loop(0, CH, step=16)
        def _(j):
            s16 = gidx_v[c, pl.ds(j, 16)]
            d16 = sidx_v[c, pl.ds(j, 16)]
            ev = plsc.load_gather(es_v, [s16])
            edv = plsc.load_gather(ed_v, [d16])
            neg = (ev + edv) < 0.0
            offs = jnp.where(neg, NP, 0).astype(jnp.int32)
            gidx_v[c, pl.ds(j, 16)] = s16 + offs
            sidx_v[c, pl.ds(j, 16)] = d16 + offs
            sval_v[c, pl.ds(j, 16)] = jnp.exp(jnp.where(neg, 0.2, 1.0) * ev)

    plsc.subcore_barrier()

    def _wait_gather(b):
        pltpu.make_async_copy(vtab_hbm.at[gidx_v.at[0]], bufs.at[b],
                              gsems[b]).wait()

    def _wait_scatter(b):
        pltpu.make_async_copy(bufs.at[b], acc.at[sidx_v.at[0]], ssems[b]).wait()
        pltpu.make_async_copy(sval_v.at[0], s_acc.at[sidx_v.at[0]],
                              ssems[b]).wait()

    for b in range(NBUF):
        pltpu.async_copy(vtab_hbm.at[gidx_v.at[b]], bufs.at[b], gsems[b])

    @pl.loop(0, NCH, step=NBUF)
    def _(c0):
        for b in range(NBUF):
            cc = c0 + b
            _wait_gather(b)
            pltpu.async_copy(bufs.at[b], acc.at[sidx_v.at[cc]], ssems[b],
                             add=True)
            pltpu.async_copy(sval_v.at[cc], s_acc.at[sidx_v.at[cc]],
                             ssems[b], add=True)

            @pl.when(cc + NBUF < NCH)
            def _():
                _wait_scatter(b)
                pltpu.async_copy(vtab_hbm.at[gidx_v.at[cc + NBUF]], bufs.at[b],
                                 gsems[b])

    for b in range(NBUF):
        _wait_scatter(b)

    plsc.subcore_barrier()

    pltpu.sync_copy(acc.at[pl.ds(base2, ROWS2_PER_TILE)],
                    n_out.at[cid, pl.ds(base2, ROWS2_PER_TILE)])

    @pl.when((sid == 0) & (cid == 0))
    def _():
        pltpu.sync_copy(s_acc, s0_out)

    @pl.when((sid == 0) & (cid == 1))
    def _():
        pltpu.sync_copy(s_acc, s1_out)


def _prep_edges(ei):
    pad = EPAD - E
    src = jnp.concatenate(
        [ei[0], (jnp.arange(pad, dtype=jnp.int32) % N_NODES)])
    dst = jnp.concatenate(
        [ei[1], N_NODES + (jnp.arange(pad, dtype=jnp.int32) % 8)])
    return src.reshape(N_TILES, NCH, CH), dst.reshape(N_TILES, NCH, CH)


def kernel(x_user, x_item, edge_uu, edge_ui, edge_iu,
           W_0_uu, as_0_uu, ad_0_uu, W_0_ui, as_0_ui, ad_0_ui,
           W_0_iu, as_0_iu, ad_0_iu, W_1_uu, as_1_uu, ad_1_uu,
           W_1_ui, as_1_ui, ad_1_ui, W_1_iu, as_1_iu, ad_1_iu,
           Wc, bc):
    suu, duu = _prep_edges(edge_uu)
    sui, dui = _prep_edges(edge_ui)
    siu, diu = _prep_edges(edge_iu)

    a0 = jnp.stack([as_0_uu, ad_0_uu, as_0_ui, ad_0_ui, as_0_iu, ad_0_iu])
    a1 = jnp.stack([as_1_uu, ad_1_uu, as_1_ui, ad_1_ui, as_1_iu, ad_1_iu])

    zn = jnp.zeros((NP2, DIM), jnp.bfloat16)
    zs = jnp.zeros((NP2,), jnp.float32)

    vuu, vui, viu, ee0 = _dense0(x_user, x_item, W_0_uu, W_0_ui, W_0_iu, a0)
    nuu, suu0, suu1 = _rel_edges(vuu, ee0[0], ee0[1], suu, duu, zn, zs)
    nui, sui0, sui1 = _rel_edges(vui, ee0[2], ee0[3], sui, dui, zn, zs)
    niu, siu0, siu1 = _rel_edges(viu, ee0[4], ee0[5], siu, diu, zn, zs)

    xu1, xi1 = _combine_elu(
        nuu, suu0, suu1, nui, sui0, sui1, niu, siu0, siu1, ee0)
    vuu, vui, viu, ee1 = _dense0(xu1, xi1, W_1_uu, W_1_ui, W_1_iu, a1)
    nuu, suu0, suu1 = _rel_edges(vuu, ee1[0], ee1[1], suu, duu, zn, zs)
    nui, sui0, sui1 = _rel_edges(vui, ee1[2], ee1[3], sui, dui, zn, zs)
    niu, siu0, siu1 = _rel_edges(viu, ee1[4], ee1[5], siu, diu, zn, zs)

    return _final(nuu, suu0, suu1, nui, sui0, sui1, niu, siu0, siu1, ee1,
                  Wc, bc)
